# baseline (speedup 1.0000x reference)
import jax
import jax.numpy as jnp
from jax import lax
from jax.experimental import pallas as pl
from jax.experimental.pallas import tpu as pltpu
from jax.experimental.pallas import tpu_sc as plsc

NC = 2
NS = 16
NT = NC * NS
LANES = 16
IDXW = 128


def _pick_j(r_base, cap):
    for d in range(min(cap, max(r_base, 1)), 0, -1):
        if r_base % d == 0:
            return d
    return 1


def _splat(ref, row):
    return plsc.load_gather(ref, [jnp.full((LANES,), row, jnp.int32)])


def _rsqrt16(x):
    i = plsc.bitcast(x, jnp.int32)
    y = plsc.bitcast(jnp.int32(0x5F3759DF) - (i >> 1), jnp.float32)
    for _ in range(3):
        y = y * (1.5 - 0.5 * x * y * y)
    return y


def _clip16(x):
    return jnp.maximum(x, 1e-6)


def _hist_call(NP, ROWS, R, EXTRA, J):
    mesh = plsc.VectorSubcoreMesh(core_axis_name="c", subcore_axis_name="s")
    sl = NP // NS
    nfull = R // J
    tail = R % J

    def body(hidx, dv_out, de_out, dv_sp, de_sp, ones_v, zbuf, niv, eiv,
             sem_a, sem_b, sem_i):
        cid = lax.axis_index("c")
        sid = lax.axis_index("s")
        w = cid * NS + sid

        def fill_ones(i, c):
            ones_v[pl.ds(i * LANES, LANES)] = jnp.ones((LANES,), jnp.float32)
            return c

        lax.fori_loop(0, IDXW // LANES, fill_ones, 0)

        def fill_zero(i, c):
            zbuf[pl.ds(i * LANES, LANES)] = jnp.zeros((LANES,), jnp.float32)
            return c

        lax.fori_loop(0, sl // LANES, fill_zero, 0)

        pltpu.sync_copy(zbuf, dv_sp.at[pl.ds(sid * sl, sl)])
        pltpu.sync_copy(zbuf, de_sp.at[pl.ds(sid * sl, sl)])
        plsc.subcore_barrier()

        def outer(b, c):
            base = w * R + b * J
            ia = pltpu.async_copy(hidx.at[0, pl.ds(base, J)], niv, sem_i)
            ib = pltpu.async_copy(hidx.at[1, pl.ds(base, J)], eiv, sem_i)
            ia.wait()
            ib.wait()

            def fire(j, c2):
                pltpu.async_copy(ones_v, dv_sp.at[niv.at[j]], sem_a, add=True)
                pltpu.async_copy(ones_v, de_sp.at[eiv.at[j]], sem_b, add=True)
                return c2

            lax.fori_loop(0, J, fire, 0)

            def drain(j, c2):
                pltpu.make_async_copy(ones_v, dv_sp.at[niv.at[j]], sem_a).wait()
                pltpu.make_async_copy(ones_v, de_sp.at[eiv.at[j]], sem_b).wait()
                return c2

            lax.fori_loop(0, J, drain, 0)
            return c

        lax.fori_loop(0, nfull, outer, 0)

        if tail:
            def tail_body(t, c):
                row = w * R + nfull * J + t
                pltpu.sync_copy(hidx.at[0, pl.ds(row, 1)], niv.at[pl.ds(0, 1)])
                pltpu.sync_copy(hidx.at[1, pl.ds(row, 1)], eiv.at[pl.ds(0, 1)])
                pltpu.sync_copy(ones_v, dv_sp.at[niv.at[0]], add=True)
                pltpu.sync_copy(ones_v, de_sp.at[eiv.at[0]], add=True)
                return c

            lax.fori_loop(0, tail, tail_body, 0)

        if EXTRA:
            @pl.when(w < EXTRA)
            def _extra():
                row = NT * R + w
                pltpu.sync_copy(hidx.at[0, pl.ds(row, 1)], niv.at[pl.ds(0, 1)])
                pltpu.sync_copy(hidx.at[1, pl.ds(row, 1)], eiv.at[pl.ds(0, 1)])
                pltpu.sync_copy(ones_v, dv_sp.at[niv.at[0]], add=True)
                pltpu.sync_copy(ones_v, de_sp.at[eiv.at[0]], add=True)

        plsc.subcore_barrier()
        pltpu.sync_copy(dv_sp.at[pl.ds(sid * sl, sl)], zbuf)
        pltpu.sync_copy(zbuf, dv_out.at[pl.ds(cid * NP + sid * sl, sl)])
        pltpu.sync_copy(de_sp.at[pl.ds(sid * sl, sl)], zbuf)
        pltpu.sync_copy(zbuf, de_out.at[pl.ds(cid * NP + sid * sl, sl)])

    return pl.kernel(
        body,
        out_type=[
            jax.ShapeDtypeStruct((NC * NP,), jnp.float32),
            jax.ShapeDtypeStruct((NC * NP,), jnp.float32),
        ],
        mesh=mesh,
        scratch_types=[
            pltpu.VMEM_SHARED((NP,), jnp.float32),
            pltpu.VMEM_SHARED((NP,), jnp.float32),
            pltpu.VMEM((IDXW,), jnp.float32),
            pltpu.VMEM((sl,), jnp.float32),
            pltpu.VMEM((J, IDXW), jnp.int32),
            pltpu.VMEM((J, IDXW), jnp.int32),
            pltpu.SemaphoreType.DMA,
            pltpu.SemaphoreType.DMA,
            pltpu.SemaphoreType.DMA,
        ],
        compiler_params=pltpu.CompilerParams(
            use_tc_tiling_on_sc=False, needs_layout_passes=False),
    )


def _scatter_call(N, NP, K, ROWS, R, EXTRA, J):
    mesh = plsc.VectorSubcoreMesh(core_axis_name="c", subcore_axis_name="s")
    sl = NP // NS
    ZR = sl // 16
    nfull = R // J
    tail = R % J
    pairs = nfull // 2
    odd = nfull % 2
    CHT = ((N + NT - 1) // NT + LANES - 1) // LANES * LANES

    def body(dv_hbm, z_hbm, hidx, out_hbm, zn_hbm, acc_sp, zrow, nia, eia,
             nib, eib, buf_a, buf_b, rsb, dva, dvb, sga, sgb, ssa, ssb, sgi):
        cid = lax.axis_index("c")
        sid = lax.axis_index("s")
        w = cid * NS + sid

        def fill_zero(i, c):
            zrow[i] = jnp.zeros((LANES,), jnp.float32)
            return c

        lax.fori_loop(0, ZR, fill_zero, 0)
        for r in range(16):
            pltpu.sync_copy(zrow, acc_sp.at[pl.ds(sid * sl + r * ZR, ZR)])

        for h in range(2):
            start = pl.multiple_of(
                jnp.minimum(sid * 2 * CHT + h * CHT, N - CHT), 8)
            pltpu.sync_copy(z_hbm.at[pl.ds(start, CHT)],
                            buf_a.at[pl.ds(0, CHT)])
            pltpu.sync_copy(dv_hbm.at[pl.ds(start, CHT)], dva)
            pltpu.sync_copy(dv_hbm.at[pl.ds(NP + start, CHT)], dvb)

            def rgrp(g, c):
                s = pl.ds(g * LANES, LANES)
                rsb[s] = _rsqrt16(_clip16(dva[s] + dvb[s]))
                return c

            lax.fori_loop(0, CHT // LANES, rgrp, 0)

            def ngrp(g, c):
                for r in range(LANES):
                    row = g * LANES + r
                    buf_a[row] = buf_a[row] * _splat(rsb, row)
                return c

            lax.fori_loop(0, CHT // LANES, ngrp, 0)
            pltpu.sync_copy(buf_a.at[pl.ds(0, CHT)],
                            zn_hbm.at[cid, pl.ds(start, CHT)])
        plsc.subcore_barrier()

        zn_c = zn_hbm.at[cid]

        def stage_idx(base, niv, eiv):
            pltpu.async_copy(hidx.at[0, pl.ds(base, J)], niv, sgi)
            pltpu.async_copy(hidx.at[1, pl.ds(base, J)], eiv, sgi)

        def wait_idx(base, niv, eiv):
            pltpu.make_async_copy(hidx.at[0, pl.ds(base, J)], niv, sgi).wait()
            pltpu.make_async_copy(hidx.at[1, pl.ds(base, J)], eiv, sgi).wait()

        def do_block(base, niv, eiv, buf, sg):
            def fire_g(j, c):
                pltpu.async_copy(zn_c.at[niv.at[j]],
                                 buf.at[pl.ds(j * IDXW, IDXW)], sg)
                return c

            lax.fori_loop(0, J, fire_g, 0)

        def drain_g_fire_s(niv, eiv, buf, sg, ss):
            def step(j, c):
                pltpu.make_async_copy(zn_c.at[niv.at[j]],
                                      buf.at[pl.ds(j * IDXW, IDXW)], sg).wait()
                pltpu.async_copy(buf.at[pl.ds(j * IDXW, IDXW)],
                                 acc_sp.at[eiv.at[j]], ss, add=True)
                return c

            lax.fori_loop(0, J, step, 0)

        def drain_s(eiv, buf, ss):
            def d(j, c):
                pltpu.make_async_copy(buf.at[pl.ds(j * IDXW, IDXW)],
                                      acc_sp.at[eiv.at[j]], ss).wait()
                return c

            lax.fori_loop(0, J, d, 0)

        def pair_body(h, c):
            base0 = w * R + (2 * h) * J
            stage_idx(base0, nia, eia)
            stage_idx(base0 + J, nib, eib)
            wait_idx(base0, nia, eia)
            do_block(base0, nia, eia, buf_a, sga)
            wait_idx(base0 + J, nib, eib)
            do_block(base0 + J, nib, eib, buf_b, sgb)
            drain_g_fire_s(nia, eia, buf_a, sga, ssa)
            drain_g_fire_s(nib, eib, buf_b, sgb, ssb)
            drain_s(eia, buf_a, ssa)
            drain_s(eib, buf_b, ssb)
            return c

        lax.fori_loop(0, pairs, pair_body, 0)

        if odd:
            base0 = w * R + (2 * pairs) * J
            stage_idx(base0, nia, eia)
            wait_idx(base0, nia, eia)
            do_block(base0, nia, eia, buf_a, sga)
            drain_g_fire_s(nia, eia, buf_a, sga, ssa)
            drain_s(eia, buf_a, ssa)

        def one_row(row):
            pltpu.sync_copy(hidx.at[0, pl.ds(row, 1)], nia.at[pl.ds(0, 1)])
            pltpu.sync_copy(hidx.at[1, pl.ds(row, 1)], eia.at[pl.ds(0, 1)])
            pltpu.async_copy(zn_c.at[nia.at[0]],
                             buf_a.at[pl.ds(0, IDXW)], sga).wait()
            pltpu.sync_copy(buf_a.at[pl.ds(0, IDXW)],
                            acc_sp.at[eia.at[0]], add=True)

        if tail:
            def tail_body(t, c):
                one_row(w * R + nfull * J + t)
                return c

            lax.fori_loop(0, tail, tail_body, 0)

        if EXTRA:
            @pl.when(w < EXTRA)
            def _extra():
                one_row(NT * R + w)

        plsc.subcore_barrier()
        for r in range(16):
            pltpu.sync_copy(acc_sp.at[pl.ds(sid * sl + r * ZR, ZR)], zrow)
            pltpu.sync_copy(zrow, out_hbm.at[cid, pl.ds(sid * sl + r * ZR, ZR)])

    return pl.kernel(
        body,
        out_type=[
            jax.ShapeDtypeStruct((NC, NP, K), jnp.float32),
            jax.ShapeDtypeStruct((NC, NP, K), jnp.float32),
        ],
        mesh=mesh,
        scratch_types=[
            pltpu.VMEM_SHARED((NP, K), jnp.float32),
            pltpu.VMEM((ZR, K), jnp.float32),
            pltpu.VMEM((J, IDXW), jnp.int32),
            pltpu.VMEM((J, IDXW), jnp.int32),
            pltpu.VMEM((J, IDXW), jnp.int32),
            pltpu.VMEM((J, IDXW), jnp.int32),
            pltpu.VMEM((J * IDXW, K), jnp.float32),
            pltpu.VMEM((J * IDXW, K), jnp.float32),
            pltpu.VMEM((CHT,), jnp.float32),
            pltpu.VMEM((CHT,), jnp.float32),
            pltpu.VMEM((CHT,), jnp.float32),
            pltpu.SemaphoreType.DMA,
            pltpu.SemaphoreType.DMA,
            pltpu.SemaphoreType.DMA,
            pltpu.SemaphoreType.DMA,
            pltpu.SemaphoreType.DMA,
        ],
        compiler_params=pltpu.CompilerParams(
            use_tc_tiling_on_sc=False, needs_layout_passes=False),
    )


def _reduce_call(N, NP, K):
    mesh = plsc.VectorSubcoreMesh(core_axis_name="c", subcore_axis_name="s")
    CHT = ((N + NT - 1) // NT + LANES - 1) // LANES * LANES

    def body(wse, dv_hbm, de_hbm, z_hbm, out_h, w0b, w1b, zb, dib, dcb,
             tm0, tm1, accv, idx32, part_sp):
        cid = lax.axis_index("c")
        sid = lax.axis_index("s")
        w = cid * NS + sid

        for i in range(2):
            accv[pl.ds(i * LANES, LANES)] = jnp.zeros((LANES,), jnp.float32)
            idx32[pl.ds(i * LANES, LANES)] = (
                lax.iota(jnp.int32, LANES) + i * LANES)

        @pl.when(sid == 0)
        def _zero_part():
            pltpu.sync_copy(accv, part_sp)
        plsc.subcore_barrier()

        start = pl.multiple_of(
            jnp.maximum(jnp.minimum(w * CHT, N - CHT), 0), 8)
        off = w * CHT - start
        count = jnp.clip(N - w * CHT, 0, CHT)

        pltpu.sync_copy(wse.at[0, pl.ds(start, CHT)], w0b)
        pltpu.sync_copy(wse.at[1, pl.ds(start, CHT)], w1b)
        pltpu.sync_copy(z_hbm.at[pl.ds(start, CHT)], zb)
        pltpu.sync_copy(dv_hbm.at[pl.ds(start, CHT)], dib)
        pltpu.sync_copy(dv_hbm.at[pl.ds(NP + start, CHT)], tm0)
        pltpu.sync_copy(de_hbm.at[pl.ds(start, CHT)], dcb)
        pltpu.sync_copy(de_hbm.at[pl.ds(NP + start, CHT)], tm1)

        def dgrp2(g, c):
            s = pl.ds(g * LANES, LANES)
            dv16 = _clip16(dib[s] + tm0[s])
            r16 = _rsqrt16(_clip16(dcb[s] + tm1[s]))
            dib[s] = r16 * r16
            dcb[s] = dv16
            return c

        lax.fori_loop(0, CHT // LANES, dgrp2, 0)

        def grp(g, carry):
            th, fd = carry
            base_row = off + g * LANES
            for r in range(LANES):
                row = base_row + r
                wr = w0b[row] + w1b[row]
                th = th + wr * wr * _splat(dib, row)
                zr = zb[row]
                fd = fd + zr * zr * _splat(dcb, row)
            return th, fd

        th, fd = lax.fori_loop(
            0, count // LANES, grp,
            (jnp.zeros((LANES,), jnp.float32), jnp.zeros((LANES,), jnp.float32)))
        accv[pl.ds(0, LANES)] = th
        accv[pl.ds(LANES, LANES)] = fd
        pltpu.sync_copy(accv, part_sp.at[idx32], add=True)
        plsc.subcore_barrier()

        @pl.when(sid == 0)
        def _out():
            pltpu.sync_copy(part_sp, accv)
            pltpu.sync_copy(accv, out_h.at[pl.ds(cid * 2 * K, 2 * K)])

    return pl.kernel(
        body,
        out_type=jax.ShapeDtypeStruct((NC * 2 * K,), jnp.float32),
        mesh=mesh,
        scratch_types=[
            pltpu.VMEM((CHT, K), jnp.float32),
            pltpu.VMEM((CHT, K), jnp.float32),
            pltpu.VMEM((CHT, K), jnp.float32),
            pltpu.VMEM((CHT,), jnp.float32),
            pltpu.VMEM((CHT,), jnp.float32),
            pltpu.VMEM((CHT,), jnp.float32),
            pltpu.VMEM((CHT,), jnp.float32),
            pltpu.VMEM((2 * K,), jnp.float32),
            pltpu.VMEM((2 * K,), jnp.int32),
            pltpu.VMEM_SHARED((2 * K,), jnp.float32),
        ],
        compiler_params=pltpu.CompilerParams(
            use_tc_tiling_on_sc=False, needs_layout_passes=False),
    )


def _final_call(K):
    def body(p, out):
        v = p[...]
        theta = v[0:1, :] + v[2:3, :]
        fdvf = jnp.clip(v[1:2, :] + v[3:4, :], 1e-6, None)
        rq = 1.0 - theta / fdvf
        rq = jnp.where(jnp.isnan(rq) | jnp.isinf(rq), 0.0, rq)
        out[...] = jnp.mean(rq)[None, None]

    return pl.pallas_call(
        body,
        out_shape=jax.ShapeDtypeStruct((1, 1), jnp.float32),
    )


def kernel(Z, hyperedge_index, num_nodes):
    N, K = Z.shape
    E = hyperedge_index.shape[1]

    NP = ((N + 1 + 2047) // 2048) * 2048

    if E % IDXW:
        epad = IDXW - E % IDXW
        hidx = jnp.concatenate(
            [hyperedge_index, jnp.full((2, epad), N, jnp.int32)], axis=1)
    else:
        hidx = hyperedge_index
    ROWS = hidx.shape[1] // IDXW
    hidx = hidx.reshape(2, ROWS, IDXW)
    R = ROWS // NT
    EXTRA = ROWS % NT
    JH = _pick_j(R, 40)
    JS = _pick_j(R, 13)

    dv_all, de_all = _hist_call(NP, ROWS, R, EXTRA, JH)(hidx)
    wse, _zn = _scatter_call(N, NP, K, ROWS, R, EXTRA, JS)(dv_all, Z, hidx)
    parts = _reduce_call(N, NP, K)(wse, dv_all, de_all, Z)
    loss = _final_call(K)(parts.reshape(NC * 2, K))
    return loss[0, 0]


# speedup vs baseline: 1.0659x; 1.0659x over previous
import jax
import jax.numpy as jnp
from jax import lax
from jax.experimental import pallas as pl
from jax.experimental.pallas import tpu as pltpu
from jax.experimental.pallas import tpu_sc as plsc

NC = 2
NS = 16
NT = NC * NS
LANES = 16
IDXW = 128


def _pick_j(r_base, cap):
    for d in range(min(cap, max(r_base, 1)), 0, -1):
        if r_base % d == 0:
            return d
    return 1


def _splat(ref, row):
    return plsc.load_gather(ref, [jnp.full((LANES,), row, jnp.int32)])


def _hist_call(NP, ROWS, R, EXTRA, J):
    mesh = plsc.VectorSubcoreMesh(core_axis_name="c", subcore_axis_name="s")
    sl = NP // NS
    nfull = R // J
    tail = R % J

    def body(hidx, dv_out, de_out, dv_sp, de_sp, ones_v, zbuf, niv, eiv,
             niv2, eiv2, sem_a, sem_b, sem_i, sem_i2):
        cid = lax.axis_index("c")
        sid = lax.axis_index("s")
        w = cid * NS + sid

        def fill_ones(i, c):
            ones_v[pl.ds(i * LANES, LANES)] = jnp.ones((LANES,), jnp.float32)
            return c

        lax.fori_loop(0, IDXW // LANES, fill_ones, 0)

        def fill_zero(i, c):
            zbuf[pl.ds(i * LANES, LANES)] = jnp.zeros((LANES,), jnp.float32)
            return c

        lax.fori_loop(0, sl // LANES, fill_zero, 0)

        pltpu.sync_copy(zbuf, dv_sp.at[pl.ds(sid * sl, sl)])
        pltpu.sync_copy(zbuf, de_sp.at[pl.ds(sid * sl, sl)])
        plsc.subcore_barrier()

        def stage(b, niv_, eiv_, sem_):
            base = w * R + b * J
            pltpu.async_copy(hidx.at[0, pl.ds(base, J)], niv_, sem_)
            pltpu.async_copy(hidx.at[1, pl.ds(base, J)], eiv_, sem_)

        def wait_stage(b, niv_, eiv_, sem_):
            base = w * R + b * J
            pltpu.make_async_copy(hidx.at[0, pl.ds(base, J)], niv_,
                                  sem_).wait()
            pltpu.make_async_copy(hidx.at[1, pl.ds(base, J)], eiv_,
                                  sem_).wait()

        def adds(niv_, eiv_):
            def fire(j, c2):
                pltpu.async_copy(ones_v, dv_sp.at[niv_.at[j]], sem_a, add=True)
                pltpu.async_copy(ones_v, de_sp.at[eiv_.at[j]], sem_b, add=True)
                return c2

            lax.fori_loop(0, J, fire, 0)

            def drain(j, c2):
                pltpu.make_async_copy(ones_v, dv_sp.at[niv_.at[j]],
                                      sem_a).wait()
                pltpu.make_async_copy(ones_v, de_sp.at[eiv_.at[j]],
                                      sem_b).wait()
                return c2

            lax.fori_loop(0, J, drain, 0)

        if nfull:
            stage(0, niv, eiv, sem_i)

        def outer2(h, c):
            b0 = 2 * h

            @pl.when(b0 + 1 < nfull)
            def _pre_b():
                stage(b0 + 1, niv2, eiv2, sem_i2)

            wait_stage(b0, niv, eiv, sem_i)
            adds(niv, eiv)

            @pl.when(b0 + 1 < nfull)
            def _do_b():
                @pl.when(b0 + 2 < nfull)
                def _pre_a():
                    stage(b0 + 2, niv, eiv, sem_i)

                wait_stage(b0 + 1, niv2, eiv2, sem_i2)
                adds(niv2, eiv2)

            return c

        lax.fori_loop(0, (nfull + 1) // 2, outer2, 0)

        if tail:
            def tail_body(t, c):
                row = w * R + nfull * J + t
                pltpu.sync_copy(hidx.at[0, pl.ds(row, 1)], niv.at[pl.ds(0, 1)])
                pltpu.sync_copy(hidx.at[1, pl.ds(row, 1)], eiv.at[pl.ds(0, 1)])
                pltpu.sync_copy(ones_v, dv_sp.at[niv.at[0]], add=True)
                pltpu.sync_copy(ones_v, de_sp.at[eiv.at[0]], add=True)
                return c

            lax.fori_loop(0, tail, tail_body, 0)

        if EXTRA:
            @pl.when(w < EXTRA)
            def _extra():
                row = NT * R + w
                pltpu.sync_copy(hidx.at[0, pl.ds(row, 1)], niv.at[pl.ds(0, 1)])
                pltpu.sync_copy(hidx.at[1, pl.ds(row, 1)], eiv.at[pl.ds(0, 1)])
                pltpu.sync_copy(ones_v, dv_sp.at[niv.at[0]], add=True)
                pltpu.sync_copy(ones_v, de_sp.at[eiv.at[0]], add=True)

        plsc.subcore_barrier()
        pltpu.sync_copy(dv_sp.at[pl.ds(sid * sl, sl)], zbuf)
        pltpu.sync_copy(zbuf, dv_out.at[pl.ds(cid * NP + sid * sl, sl)])
        pltpu.sync_copy(de_sp.at[pl.ds(sid * sl, sl)], zbuf)
        pltpu.sync_copy(zbuf, de_out.at[pl.ds(cid * NP + sid * sl, sl)])

    return pl.kernel(
        body,
        out_type=[
            jax.ShapeDtypeStruct((NC * NP,), jnp.float32),
            jax.ShapeDtypeStruct((NC * NP,), jnp.float32),
        ],
        mesh=mesh,
        scratch_types=[
            pltpu.VMEM_SHARED((NP,), jnp.float32),
            pltpu.VMEM_SHARED((NP,), jnp.float32),
            pltpu.VMEM((IDXW,), jnp.float32),
            pltpu.VMEM((sl,), jnp.float32),
            pltpu.VMEM((J, IDXW), jnp.int32),
            pltpu.VMEM((J, IDXW), jnp.int32),
            pltpu.VMEM((J, IDXW), jnp.int32),
            pltpu.VMEM((J, IDXW), jnp.int32),
            pltpu.SemaphoreType.DMA,
            pltpu.SemaphoreType.DMA,
            pltpu.SemaphoreType.DMA,
            pltpu.SemaphoreType.DMA,
        ],
        compiler_params=pltpu.CompilerParams(
            use_tc_tiling_on_sc=False, needs_layout_passes=False),
    )


def _rs_call(NPR):
    def body(dva, dea, rs, dvc, deinv):
        dv = jnp.clip(dva[0:NPR, :] + dva[NPR:2 * NPR, :], 1e-6, None)
        dvc[...] = dv
        rs[...] = lax.rsqrt(dv)
        deinv[...] = 1.0 / jnp.clip(
            dea[0:NPR, :] + dea[NPR:2 * NPR, :], 1e-6, None)

    return pl.pallas_call(
        body,
        out_shape=[
            jax.ShapeDtypeStruct((NPR, 128), jnp.float32),
            jax.ShapeDtypeStruct((NPR, 128), jnp.float32),
            jax.ShapeDtypeStruct((NPR, 128), jnp.float32),
        ],
    )


def _scatter_call(N, NP, K, ROWS, R, EXTRA, J):
    mesh = plsc.VectorSubcoreMesh(core_axis_name="c", subcore_axis_name="s")
    sl = NP // NS
    ZR = sl // 16
    nfull = R // J
    tail = R % J
    pairs = nfull // 2
    odd = nfull % 2
    CHT = ((N + NT - 1) // NT + LANES - 1) // LANES * LANES

    def body(rs_hbm, z_hbm, hidx, out_hbm, zn_hbm, acc_sp, zrow, nia, eia,
             nib, eib, buf_a, buf_b, rsb, sga, sgb, ssa, ssb, sgi):
        cid = lax.axis_index("c")
        sid = lax.axis_index("s")
        w = cid * NS + sid

        def fill_zero(i, c):
            zrow[i] = jnp.zeros((LANES,), jnp.float32)
            return c

        lax.fori_loop(0, ZR, fill_zero, 0)
        for r in range(16):
            pltpu.sync_copy(zrow, acc_sp.at[pl.ds(sid * sl + r * ZR, ZR)])

        for h in range(2):
            start = pl.multiple_of(
                jnp.minimum(sid * 2 * CHT + h * CHT, N - CHT), 8)
            pltpu.sync_copy(z_hbm.at[pl.ds(start, CHT)],
                            buf_a.at[pl.ds(0, CHT)])
            pltpu.sync_copy(rs_hbm.at[pl.ds(start, CHT)], rsb)

            def ngrp(g, c):
                for r in range(LANES):
                    row = g * LANES + r
                    buf_a[row] = buf_a[row] * _splat(rsb, row)
                return c

            lax.fori_loop(0, CHT // LANES, ngrp, 0)
            pltpu.sync_copy(buf_a.at[pl.ds(0, CHT)],
                            zn_hbm.at[cid, pl.ds(start, CHT)])
        plsc.subcore_barrier()

        zn_c = zn_hbm.at[cid]

        def stage_idx(base, niv, eiv):
            pltpu.async_copy(hidx.at[0, pl.ds(base, J)], niv, sgi)
            pltpu.async_copy(hidx.at[1, pl.ds(base, J)], eiv, sgi)

        def wait_idx(base, niv, eiv):
            pltpu.make_async_copy(hidx.at[0, pl.ds(base, J)], niv, sgi).wait()
            pltpu.make_async_copy(hidx.at[1, pl.ds(base, J)], eiv, sgi).wait()

        def do_block(base, niv, eiv, buf, sg):
            def fire_g(j, c):
                pltpu.async_copy(zn_c.at[niv.at[j]],
                                 buf.at[pl.ds(j * IDXW, IDXW)], sg)
                return c

            lax.fori_loop(0, J, fire_g, 0)

        def drain_g_fire_s(niv, eiv, buf, sg, ss):
            def step(j, c):
                pltpu.make_async_copy(zn_c.at[niv.at[j]],
                                      buf.at[pl.ds(j * IDXW, IDXW)], sg).wait()
                pltpu.async_copy(buf.at[pl.ds(j * IDXW, IDXW)],
                                 acc_sp.at[eiv.at[j]], ss, add=True)
                return c

            lax.fori_loop(0, J, step, 0)

        def drain_s(eiv, buf, ss):
            def d(j, c):
                pltpu.make_async_copy(buf.at[pl.ds(j * IDXW, IDXW)],
                                      acc_sp.at[eiv.at[j]], ss).wait()
                return c

            lax.fori_loop(0, J, d, 0)

        def pair_body(h, c):
            base0 = w * R + (2 * h) * J
            stage_idx(base0, nia, eia)
            stage_idx(base0 + J, nib, eib)
            wait_idx(base0, nia, eia)
            do_block(base0, nia, eia, buf_a, sga)
            wait_idx(base0 + J, nib, eib)
            do_block(base0 + J, nib, eib, buf_b, sgb)
            drain_g_fire_s(nia, eia, buf_a, sga, ssa)
            drain_g_fire_s(nib, eib, buf_b, sgb, ssb)
            drain_s(eia, buf_a, ssa)
            drain_s(eib, buf_b, ssb)
            return c

        lax.fori_loop(0, pairs, pair_body, 0)

        if odd:
            base0 = w * R + (2 * pairs) * J
            stage_idx(base0, nia, eia)
            wait_idx(base0, nia, eia)
            do_block(base0, nia, eia, buf_a, sga)
            drain_g_fire_s(nia, eia, buf_a, sga, ssa)
            drain_s(eia, buf_a, ssa)

        def one_row(row):
            pltpu.sync_copy(hidx.at[0, pl.ds(row, 1)], nia.at[pl.ds(0, 1)])
            pltpu.sync_copy(hidx.at[1, pl.ds(row, 1)], eia.at[pl.ds(0, 1)])
            pltpu.async_copy(zn_c.at[nia.at[0]],
                             buf_a.at[pl.ds(0, IDXW)], sga).wait()
            pltpu.sync_copy(buf_a.at[pl.ds(0, IDXW)],
                            acc_sp.at[eia.at[0]], add=True)

        if tail:
            def tail_body(t, c):
                one_row(w * R + nfull * J + t)
                return c

            lax.fori_loop(0, tail, tail_body, 0)

        if EXTRA:
            @pl.when(w < EXTRA)
            def _extra():
                one_row(NT * R + w)

        plsc.subcore_barrier()
        for r in range(16):
            pltpu.sync_copy(acc_sp.at[pl.ds(sid * sl + r * ZR, ZR)], zrow)
            pltpu.sync_copy(zrow, out_hbm.at[cid, pl.ds(sid * sl + r * ZR, ZR)])

    return pl.kernel(
        body,
        out_type=[
            jax.ShapeDtypeStruct((NC, NP, K), jnp.float32),
            jax.ShapeDtypeStruct((NC, NP, K), jnp.float32),
        ],
        mesh=mesh,
        scratch_types=[
            pltpu.VMEM_SHARED((NP, K), jnp.float32),
            pltpu.VMEM((ZR, K), jnp.float32),
            pltpu.VMEM((J, IDXW), jnp.int32),
            pltpu.VMEM((J, IDXW), jnp.int32),
            pltpu.VMEM((J, IDXW), jnp.int32),
            pltpu.VMEM((J, IDXW), jnp.int32),
            pltpu.VMEM((J * IDXW, K), jnp.float32),
            pltpu.VMEM((J * IDXW, K), jnp.float32),
            pltpu.VMEM((CHT,), jnp.float32),
            pltpu.SemaphoreType.DMA,
            pltpu.SemaphoreType.DMA,
            pltpu.SemaphoreType.DMA,
            pltpu.SemaphoreType.DMA,
            pltpu.SemaphoreType.DMA,
        ],
        compiler_params=pltpu.CompilerParams(
            use_tc_tiling_on_sc=False, needs_layout_passes=False),
    )


def _reduce_call(N, NP, K):
    mesh = plsc.VectorSubcoreMesh(core_axis_name="c", subcore_axis_name="s")
    CHT = ((N + NT - 1) // NT + LANES - 1) // LANES * LANES

    def body(wse, deinv_h, z_hbm, dvc_h, out_h, w0b, w1b, zb, dib, dcb,
             accv, idx32, part_sp):
        cid = lax.axis_index("c")
        sid = lax.axis_index("s")
        w = cid * NS + sid

        for i in range(2):
            accv[pl.ds(i * LANES, LANES)] = jnp.zeros((LANES,), jnp.float32)
            idx32[pl.ds(i * LANES, LANES)] = (
                lax.iota(jnp.int32, LANES) + i * LANES)

        @pl.when(sid == 0)
        def _zero_part():
            pltpu.sync_copy(accv, part_sp)
        plsc.subcore_barrier()

        start = pl.multiple_of(
            jnp.maximum(jnp.minimum(w * CHT, N - CHT), 0), 8)
        off = w * CHT - start
        count = jnp.clip(N - w * CHT, 0, CHT)

        pltpu.sync_copy(wse.at[0, pl.ds(start, CHT)], w0b)
        pltpu.sync_copy(wse.at[1, pl.ds(start, CHT)], w1b)
        pltpu.sync_copy(z_hbm.at[pl.ds(start, CHT)], zb)
        pltpu.sync_copy(deinv_h.at[pl.ds(start, CHT)], dib)
        pltpu.sync_copy(dvc_h.at[pl.ds(start, CHT)], dcb)

        def grp(g, carry):
            th, fd = carry
            base_row = off + g * LANES
            for r in range(LANES):
                row = base_row + r
                wr = w0b[row] + w1b[row]
                th = th + wr * wr * _splat(dib, row)
                zr = zb[row]
                fd = fd + zr * zr * _splat(dcb, row)
            return th, fd

        th, fd = lax.fori_loop(
            0, count // LANES, grp,
            (jnp.zeros((LANES,), jnp.float32), jnp.zeros((LANES,), jnp.float32)))
        accv[pl.ds(0, LANES)] = th
        accv[pl.ds(LANES, LANES)] = fd
        pltpu.sync_copy(accv, part_sp.at[idx32], add=True)
        plsc.subcore_barrier()

        @pl.when(sid == 0)
        def _out():
            pltpu.sync_copy(part_sp, accv)
            pltpu.sync_copy(accv, out_h.at[pl.ds(cid * 2 * K, 2 * K)])

    return pl.kernel(
        body,
        out_type=jax.ShapeDtypeStruct((NC * 2 * K,), jnp.float32),
        mesh=mesh,
        scratch_types=[
            pltpu.VMEM((CHT, K), jnp.float32),
            pltpu.VMEM((CHT, K), jnp.float32),
            pltpu.VMEM((CHT, K), jnp.float32),
            pltpu.VMEM((CHT,), jnp.float32),
            pltpu.VMEM((CHT,), jnp.float32),
            pltpu.VMEM((2 * K,), jnp.float32),
            pltpu.VMEM((2 * K,), jnp.int32),
            pltpu.VMEM_SHARED((2 * K,), jnp.float32),
        ],
        compiler_params=pltpu.CompilerParams(
            use_tc_tiling_on_sc=False, needs_layout_passes=False),
    )


def _final_call(K):
    def body(p, out):
        v = p[...]
        theta = v[0:1, :] + v[2:3, :]
        fdvf = jnp.clip(v[1:2, :] + v[3:4, :], 1e-6, None)
        rq = 1.0 - theta / fdvf
        rq = jnp.where(jnp.isnan(rq) | jnp.isinf(rq), 0.0, rq)
        out[...] = jnp.mean(rq)[None, None]

    return pl.pallas_call(
        body,
        out_shape=jax.ShapeDtypeStruct((1, 1), jnp.float32),
    )


def kernel(Z, hyperedge_index, num_nodes):
    N, K = Z.shape
    E = hyperedge_index.shape[1]

    NP = ((N + 1 + 2047) // 2048) * 2048
    NPR = NP // 128

    if E % IDXW:
        epad = IDXW - E % IDXW
        hidx = jnp.concatenate(
            [hyperedge_index, jnp.full((2, epad), N, jnp.int32)], axis=1)
    else:
        hidx = hyperedge_index
    ROWS = hidx.shape[1] // IDXW
    hidx = hidx.reshape(2, ROWS, IDXW)
    R = ROWS // NT
    EXTRA = ROWS % NT
    JH = _pick_j(R, 40)
    JS = _pick_j(R, 13)

    dv_all, de_all = _hist_call(NP, ROWS, R, EXTRA, JH)(hidx)
    rs, dvc, deinv = _rs_call(NPR)(
        dv_all.reshape(2 * NPR, 128), de_all.reshape(2 * NPR, 128))
    rs = rs.reshape(NP)
    dvc = dvc.reshape(NP)
    deinv = deinv.reshape(NP)
    wse, _zn = _scatter_call(N, NP, K, ROWS, R, EXTRA, JS)(rs, Z, hidx)
    parts = _reduce_call(N, NP, K)(wse, deinv, Z, dvc)
    loss = _final_call(K)(parts.reshape(NC * 2, K))
    return loss[0, 0]


# speedup vs baseline: 1.0982x; 1.0303x over previous
import jax
import jax.numpy as jnp
from jax import lax
from jax.experimental import pallas as pl
from jax.experimental.pallas import tpu as pltpu
from jax.experimental.pallas import tpu_sc as plsc

NC = 2
NS = 16
NT = NC * NS
LANES = 16
IDXW = 128


def _pick_j(r_base, cap):
    for d in range(min(cap, max(r_base, 1)), 0, -1):
        if r_base % d == 0:
            return d
    return 1


def _splat(ref, row):
    return plsc.load_gather(ref, [jnp.full((LANES,), row, jnp.int32)])


def _hist_call(NP, ROWS, R, EXTRA, J):
    mesh = plsc.VectorSubcoreMesh(core_axis_name="c", subcore_axis_name="s")
    sl = NP // NS
    nfull = R // J
    tail = R % J

    def body(hidx, dv_out, de_out, dv_sp, de_sp, ones_v, zbuf, niv, eiv,
             niv2, eiv2, sem_a, sem_b, sem_i, sem_i2):
        cid = lax.axis_index("c")
        sid = lax.axis_index("s")
        w = cid * NS + sid

        def fill_ones(i, c):
            ones_v[pl.ds(i * LANES, LANES)] = jnp.ones((LANES,), jnp.float32)
            return c

        lax.fori_loop(0, IDXW // LANES, fill_ones, 0)

        def fill_zero(i, c):
            zbuf[pl.ds(i * LANES, LANES)] = jnp.zeros((LANES,), jnp.float32)
            return c

        lax.fori_loop(0, sl // LANES, fill_zero, 0)

        pltpu.sync_copy(zbuf, dv_sp.at[pl.ds(sid * sl, sl)])
        pltpu.sync_copy(zbuf, de_sp.at[pl.ds(sid * sl, sl)])
        plsc.subcore_barrier()

        def stage(b, niv_, eiv_, sem_):
            base = w * R + b * J
            pltpu.async_copy(hidx.at[0, pl.ds(base, J)], niv_, sem_)
            pltpu.async_copy(hidx.at[1, pl.ds(base, J)], eiv_, sem_)

        def wait_stage(b, niv_, eiv_, sem_):
            base = w * R + b * J
            pltpu.make_async_copy(hidx.at[0, pl.ds(base, J)], niv_,
                                  sem_).wait()
            pltpu.make_async_copy(hidx.at[1, pl.ds(base, J)], eiv_,
                                  sem_).wait()

        def adds(niv_, eiv_):
            def fire(j, c2):
                pltpu.async_copy(ones_v, dv_sp.at[niv_.at[j]], sem_a, add=True)
                pltpu.async_copy(ones_v, de_sp.at[eiv_.at[j]], sem_b, add=True)
                return c2

            lax.fori_loop(0, J, fire, 0)

            def drain(j, c2):
                pltpu.make_async_copy(ones_v, dv_sp.at[niv_.at[j]],
                                      sem_a).wait()
                pltpu.make_async_copy(ones_v, de_sp.at[eiv_.at[j]],
                                      sem_b).wait()
                return c2

            lax.fori_loop(0, J, drain, 0)

        if nfull:
            stage(0, niv, eiv, sem_i)

        def outer2(h, c):
            b0 = 2 * h

            @pl.when(b0 + 1 < nfull)
            def _pre_b():
                stage(b0 + 1, niv2, eiv2, sem_i2)

            wait_stage(b0, niv, eiv, sem_i)
            adds(niv, eiv)

            @pl.when(b0 + 1 < nfull)
            def _do_b():
                @pl.when(b0 + 2 < nfull)
                def _pre_a():
                    stage(b0 + 2, niv, eiv, sem_i)

                wait_stage(b0 + 1, niv2, eiv2, sem_i2)
                adds(niv2, eiv2)

            return c

        lax.fori_loop(0, (nfull + 1) // 2, outer2, 0)

        if tail:
            def tail_body(t, c):
                row = w * R + nfull * J + t
                pltpu.sync_copy(hidx.at[0, pl.ds(row, 1)], niv.at[pl.ds(0, 1)])
                pltpu.sync_copy(hidx.at[1, pl.ds(row, 1)], eiv.at[pl.ds(0, 1)])
                pltpu.sync_copy(ones_v, dv_sp.at[niv.at[0]], add=True)
                pltpu.sync_copy(ones_v, de_sp.at[eiv.at[0]], add=True)
                return c

            lax.fori_loop(0, tail, tail_body, 0)

        if EXTRA:
            @pl.when(w < EXTRA)
            def _extra():
                row = NT * R + w
                pltpu.sync_copy(hidx.at[0, pl.ds(row, 1)], niv.at[pl.ds(0, 1)])
                pltpu.sync_copy(hidx.at[1, pl.ds(row, 1)], eiv.at[pl.ds(0, 1)])
                pltpu.sync_copy(ones_v, dv_sp.at[niv.at[0]], add=True)
                pltpu.sync_copy(ones_v, de_sp.at[eiv.at[0]], add=True)

        plsc.subcore_barrier()
        pltpu.sync_copy(dv_sp.at[pl.ds(sid * sl, sl)], zbuf)
        pltpu.sync_copy(zbuf, dv_out.at[pl.ds(cid * NP + sid * sl, sl)])
        pltpu.sync_copy(de_sp.at[pl.ds(sid * sl, sl)], zbuf)
        pltpu.sync_copy(zbuf, de_out.at[pl.ds(cid * NP + sid * sl, sl)])

    return pl.kernel(
        body,
        out_type=[
            jax.ShapeDtypeStruct((NC * NP,), jnp.float32),
            jax.ShapeDtypeStruct((NC * NP,), jnp.float32),
        ],
        mesh=mesh,
        scratch_types=[
            pltpu.VMEM_SHARED((NP,), jnp.float32),
            pltpu.VMEM_SHARED((NP,), jnp.float32),
            pltpu.VMEM((IDXW,), jnp.float32),
            pltpu.VMEM((sl,), jnp.float32),
            pltpu.VMEM((J, IDXW), jnp.int32),
            pltpu.VMEM((J, IDXW), jnp.int32),
            pltpu.VMEM((J, IDXW), jnp.int32),
            pltpu.VMEM((J, IDXW), jnp.int32),
            pltpu.SemaphoreType.DMA,
            pltpu.SemaphoreType.DMA,
            pltpu.SemaphoreType.DMA,
            pltpu.SemaphoreType.DMA,
        ],
        compiler_params=pltpu.CompilerParams(
            use_tc_tiling_on_sc=False, needs_layout_passes=False),
    )


def _rs_call(NPR):
    def body(dva, dea, rs, dvc, deinv):
        dv = jnp.clip(dva[0:NPR, :] + dva[NPR:2 * NPR, :], 1e-6, None)
        dvc[...] = dv
        rs[...] = lax.rsqrt(dv)
        deinv[...] = 1.0 / jnp.clip(
            dea[0:NPR, :] + dea[NPR:2 * NPR, :], 1e-6, None)

    return pl.pallas_call(
        body,
        out_shape=[
            jax.ShapeDtypeStruct((NPR, 128), jnp.float32),
            jax.ShapeDtypeStruct((NPR, 128), jnp.float32),
            jax.ShapeDtypeStruct((NPR, 128), jnp.float32),
        ],
    )


def _scatter_call(N, NP, K, ROWS, R, EXTRA, J):
    mesh = plsc.VectorSubcoreMesh(core_axis_name="c", subcore_axis_name="s")
    sl = NP // NS
    ZR = sl // 16
    nfull = R // J
    tail = R % J
    pairs = nfull // 2
    odd = nfull % 2
    CHT = ((N + NT - 1) // NT + LANES - 1) // LANES * LANES

    def body(rs_hbm, z_hbm, hidx, out_hbm, zn_hbm, acc_sp, zrow, nia, eia,
             nib, eib, nic, eic, nid, eid, buf_a, buf_b, rsb,
             sga, sgb, ssa, ssb, sgi, sgi2):
        cid = lax.axis_index("c")
        sid = lax.axis_index("s")
        w = cid * NS + sid

        def fill_zero(i, c):
            zrow[i] = jnp.zeros((LANES,), jnp.float32)
            return c

        lax.fori_loop(0, ZR, fill_zero, 0)
        for r in range(16):
            pltpu.sync_copy(zrow, acc_sp.at[pl.ds(sid * sl + r * ZR, ZR)])

        for h in range(2):
            start = pl.multiple_of(
                jnp.minimum(sid * 2 * CHT + h * CHT, N - CHT), 8)
            pltpu.sync_copy(z_hbm.at[pl.ds(start, CHT)],
                            buf_a.at[pl.ds(0, CHT)])
            pltpu.sync_copy(rs_hbm.at[pl.ds(start, CHT)], rsb)

            def ngrp(g, c):
                for r in range(LANES):
                    row = g * LANES + r
                    buf_a[row] = buf_a[row] * _splat(rsb, row)
                return c

            lax.fori_loop(0, CHT // LANES, ngrp, 0)
            pltpu.sync_copy(buf_a.at[pl.ds(0, CHT)],
                            zn_hbm.at[cid, pl.ds(start, CHT)])
        plsc.subcore_barrier()

        zn_c = zn_hbm.at[cid]

        def stage_idx(base, niv, eiv, sem_):
            pltpu.async_copy(hidx.at[0, pl.ds(base, J)], niv, sem_)
            pltpu.async_copy(hidx.at[1, pl.ds(base, J)], eiv, sem_)

        def wait_idx(base, niv, eiv, sem_):
            pltpu.make_async_copy(hidx.at[0, pl.ds(base, J)], niv, sem_).wait()
            pltpu.make_async_copy(hidx.at[1, pl.ds(base, J)], eiv, sem_).wait()

        def do_block(base, niv, eiv, buf, sg):
            def fire_g(j, c):
                pltpu.async_copy(zn_c.at[niv.at[j]],
                                 buf.at[pl.ds(j * IDXW, IDXW)], sg)
                return c

            lax.fori_loop(0, J, fire_g, 0)

        def drain_g_fire_s(niv, eiv, buf, sg, ss):
            def step(j, c):
                pltpu.make_async_copy(zn_c.at[niv.at[j]],
                                      buf.at[pl.ds(j * IDXW, IDXW)], sg).wait()
                pltpu.async_copy(buf.at[pl.ds(j * IDXW, IDXW)],
                                 acc_sp.at[eiv.at[j]], ss, add=True)
                return c

            lax.fori_loop(0, J, step, 0)

        def drain_s(eiv, buf, ss):
            def d(j, c):
                pltpu.make_async_copy(buf.at[pl.ds(j * IDXW, IDXW)],
                                      acc_sp.at[eiv.at[j]], ss).wait()
                return c

            lax.fori_loop(0, J, d, 0)

        def stage_pair(p, n0, e0, n1, e1, sem_):
            base0 = w * R + (2 * p) * J
            stage_idx(base0, n0, e0, sem_)
            stage_idx(base0 + J, n1, e1, sem_)

        def pair_proc(p, n0, e0, n1, e1, sem_):
            base0 = w * R + (2 * p) * J
            wait_idx(base0, n0, e0, sem_)
            do_block(base0, n0, e0, buf_a, sga)
            wait_idx(base0 + J, n1, e1, sem_)
            do_block(base0 + J, n1, e1, buf_b, sgb)
            drain_g_fire_s(n0, e0, buf_a, sga, ssa)
            drain_g_fire_s(n1, e1, buf_b, sgb, ssb)
            drain_s(e0, buf_a, ssa)
            drain_s(e1, buf_b, ssb)

        if pairs:
            stage_pair(0, nia, eia, nib, eib, sgi)

        def outer2(i, c):
            p0 = 2 * i

            @pl.when(p0 + 1 < pairs)
            def _pre_cd():
                stage_pair(p0 + 1, nic, eic, nid, eid, sgi2)

            pair_proc(p0, nia, eia, nib, eib, sgi)

            @pl.when(p0 + 1 < pairs)
            def _do_cd():
                @pl.when(p0 + 2 < pairs)
                def _pre_ab():
                    stage_pair(p0 + 2, nia, eia, nib, eib, sgi)

                pair_proc(p0 + 1, nic, eic, nid, eid, sgi2)

            return c

        lax.fori_loop(0, (pairs + 1) // 2, outer2, 0)

        if odd:
            base0 = w * R + (2 * pairs) * J
            stage_idx(base0, nia, eia, sgi)
            wait_idx(base0, nia, eia, sgi)
            do_block(base0, nia, eia, buf_a, sga)
            drain_g_fire_s(nia, eia, buf_a, sga, ssa)
            drain_s(eia, buf_a, ssa)

        def one_row(row):
            pltpu.sync_copy(hidx.at[0, pl.ds(row, 1)], nia.at[pl.ds(0, 1)])
            pltpu.sync_copy(hidx.at[1, pl.ds(row, 1)], eia.at[pl.ds(0, 1)])
            pltpu.async_copy(zn_c.at[nia.at[0]],
                             buf_a.at[pl.ds(0, IDXW)], sga).wait()
            pltpu.sync_copy(buf_a.at[pl.ds(0, IDXW)],
                            acc_sp.at[eia.at[0]], add=True)

        if tail:
            def tail_body(t, c):
                one_row(w * R + nfull * J + t)
                return c

            lax.fori_loop(0, tail, tail_body, 0)

        if EXTRA:
            @pl.when(w < EXTRA)
            def _extra():
                one_row(NT * R + w)

        plsc.subcore_barrier()
        for r in range(16):
            pltpu.sync_copy(acc_sp.at[pl.ds(sid * sl + r * ZR, ZR)], zrow)
            pltpu.sync_copy(zrow, out_hbm.at[cid, pl.ds(sid * sl + r * ZR, ZR)])

    return pl.kernel(
        body,
        out_type=[
            jax.ShapeDtypeStruct((NC, NP, K), jnp.float32),
            jax.ShapeDtypeStruct((NC, NP, K), jnp.float32),
        ],
        mesh=mesh,
        scratch_types=[
            pltpu.VMEM_SHARED((NP, K), jnp.float32),
            pltpu.VMEM((ZR, K), jnp.float32),
            pltpu.VMEM((J, IDXW), jnp.int32),
            pltpu.VMEM((J, IDXW), jnp.int32),
            pltpu.VMEM((J, IDXW), jnp.int32),
            pltpu.VMEM((J, IDXW), jnp.int32),
            pltpu.VMEM((J, IDXW), jnp.int32),
            pltpu.VMEM((J, IDXW), jnp.int32),
            pltpu.VMEM((J, IDXW), jnp.int32),
            pltpu.VMEM((J, IDXW), jnp.int32),
            pltpu.VMEM((J * IDXW, K), jnp.float32),
            pltpu.VMEM((J * IDXW, K), jnp.float32),
            pltpu.VMEM((CHT,), jnp.float32),
            pltpu.SemaphoreType.DMA,
            pltpu.SemaphoreType.DMA,
            pltpu.SemaphoreType.DMA,
            pltpu.SemaphoreType.DMA,
            pltpu.SemaphoreType.DMA,
            pltpu.SemaphoreType.DMA,
        ],
        compiler_params=pltpu.CompilerParams(
            use_tc_tiling_on_sc=False, needs_layout_passes=False),
    )


def _reduce_call(N, NP, K):
    mesh = plsc.VectorSubcoreMesh(core_axis_name="c", subcore_axis_name="s")
    CHT = ((N + NT - 1) // NT + LANES - 1) // LANES * LANES

    def body(wse, deinv_h, z_hbm, dvc_h, out_h, w0b, w1b, zb, dib, dcb,
             accv, idx32, part_sp):
        cid = lax.axis_index("c")
        sid = lax.axis_index("s")
        w = cid * NS + sid

        for i in range(2):
            accv[pl.ds(i * LANES, LANES)] = jnp.zeros((LANES,), jnp.float32)
            idx32[pl.ds(i * LANES, LANES)] = (
                lax.iota(jnp.int32, LANES) + i * LANES)

        @pl.when(sid == 0)
        def _zero_part():
            pltpu.sync_copy(accv, part_sp)
        plsc.subcore_barrier()

        start = pl.multiple_of(
            jnp.maximum(jnp.minimum(w * CHT, N - CHT), 0), 8)
        off = w * CHT - start
        count = jnp.clip(N - w * CHT, 0, CHT)

        pltpu.sync_copy(wse.at[0, pl.ds(start, CHT)], w0b)
        pltpu.sync_copy(wse.at[1, pl.ds(start, CHT)], w1b)
        pltpu.sync_copy(z_hbm.at[pl.ds(start, CHT)], zb)
        pltpu.sync_copy(deinv_h.at[pl.ds(start, CHT)], dib)
        pltpu.sync_copy(dvc_h.at[pl.ds(start, CHT)], dcb)

        def grp(g, carry):
            th, fd = carry
            base_row = off + g * LANES
            for r in range(LANES):
                row = base_row + r
                wr = w0b[row] + w1b[row]
                th = th + wr * wr * _splat(dib, row)
                zr = zb[row]
                fd = fd + zr * zr * _splat(dcb, row)
            return th, fd

        th, fd = lax.fori_loop(
            0, count // LANES, grp,
            (jnp.zeros((LANES,), jnp.float32), jnp.zeros((LANES,), jnp.float32)))
        accv[pl.ds(0, LANES)] = th
        accv[pl.ds(LANES, LANES)] = fd
        pltpu.sync_copy(accv, part_sp.at[idx32], add=True)
        plsc.subcore_barrier()

        @pl.when(sid == 0)
        def _out():
            pltpu.sync_copy(part_sp, accv)
            pltpu.sync_copy(accv, out_h.at[pl.ds(cid * 2 * K, 2 * K)])

    return pl.kernel(
        body,
        out_type=jax.ShapeDtypeStruct((NC * 2 * K,), jnp.float32),
        mesh=mesh,
        scratch_types=[
            pltpu.VMEM((CHT, K), jnp.float32),
            pltpu.VMEM((CHT, K), jnp.float32),
            pltpu.VMEM((CHT, K), jnp.float32),
            pltpu.VMEM((CHT,), jnp.float32),
            pltpu.VMEM((CHT,), jnp.float32),
            pltpu.VMEM((2 * K,), jnp.float32),
            pltpu.VMEM((2 * K,), jnp.int32),
            pltpu.VMEM_SHARED((2 * K,), jnp.float32),
        ],
        compiler_params=pltpu.CompilerParams(
            use_tc_tiling_on_sc=False, needs_layout_passes=False),
    )


def _final_call(K):
    def body(p, out):
        v = p[...]
        theta = v[0:1, :] + v[2:3, :]
        fdvf = jnp.clip(v[1:2, :] + v[3:4, :], 1e-6, None)
        rq = 1.0 - theta / fdvf
        rq = jnp.where(jnp.isnan(rq) | jnp.isinf(rq), 0.0, rq)
        out[...] = jnp.mean(rq)[None, None]

    return pl.pallas_call(
        body,
        out_shape=jax.ShapeDtypeStruct((1, 1), jnp.float32),
    )


def kernel(Z, hyperedge_index, num_nodes):
    N, K = Z.shape
    E = hyperedge_index.shape[1]

    NP = ((N + 1 + 2047) // 2048) * 2048
    NPR = NP // 128

    if E % IDXW:
        epad = IDXW - E % IDXW
        hidx = jnp.concatenate(
            [hyperedge_index, jnp.full((2, epad), N, jnp.int32)], axis=1)
    else:
        hidx = hyperedge_index
    ROWS = hidx.shape[1] // IDXW
    hidx = hidx.reshape(2, ROWS, IDXW)
    R = ROWS // NT
    EXTRA = ROWS % NT
    JH = _pick_j(R, 40)
    JS = _pick_j(R, 13)

    dv_all, de_all = _hist_call(NP, ROWS, R, EXTRA, JH)(hidx)
    rs, dvc, deinv = _rs_call(NPR)(
        dv_all.reshape(2 * NPR, 128), de_all.reshape(2 * NPR, 128))
    rs = rs.reshape(NP)
    dvc = dvc.reshape(NP)
    deinv = deinv.reshape(NP)
    wse, _zn = _scatter_call(N, NP, K, ROWS, R, EXTRA, JS)(rs, Z, hidx)
    parts = _reduce_call(N, NP, K)(wse, deinv, Z, dvc)
    loss = _final_call(K)(parts.reshape(NC * 2, K))
    return loss[0, 0]


# speedup vs baseline: 1.1098x; 1.0105x over previous
import jax
import jax.numpy as jnp
from jax import lax
from jax.experimental import pallas as pl
from jax.experimental.pallas import tpu as pltpu
from jax.experimental.pallas import tpu_sc as plsc

NC = 2
NS = 16
NT = NC * NS
LANES = 16
IDXW = 128


def _pick_j(r_base, cap):
    for d in range(min(cap, max(r_base, 1)), 0, -1):
        if r_base % d == 0:
            return d
    return 1


def _splat(ref, row):
    return plsc.load_gather(ref, [jnp.full((LANES,), row, jnp.int32)])


def _hist_call(NP, ROWS, R, EXTRA, J):
    mesh = plsc.VectorSubcoreMesh(core_axis_name="c", subcore_axis_name="s")
    sl = NP // NS
    nfull = R // J
    tail = R % J

    def body(hidx, dv_out, de_out, dv_sp, de_sp, ones_v, zbuf, niv, eiv,
             niv2, eiv2, sem_a, sem_b, sem_i, sem_i2):
        cid = lax.axis_index("c")
        sid = lax.axis_index("s")
        w = cid * NS + sid

        def fill_ones(i, c):
            ones_v[pl.ds(i * LANES, LANES)] = jnp.ones((LANES,), jnp.float32)
            return c

        lax.fori_loop(0, IDXW // LANES, fill_ones, 0)

        def fill_zero(i, c):
            zbuf[pl.ds(i * LANES, LANES)] = jnp.zeros((LANES,), jnp.float32)
            return c

        lax.fori_loop(0, sl // LANES, fill_zero, 0)

        pltpu.sync_copy(zbuf, dv_sp.at[pl.ds(sid * sl, sl)])
        pltpu.sync_copy(zbuf, de_sp.at[pl.ds(sid * sl, sl)])
        plsc.subcore_barrier()

        def stage(b, niv_, eiv_, sem_):
            base = w * R + b * J
            pltpu.async_copy(hidx.at[0, pl.ds(base, J)], niv_, sem_)
            pltpu.async_copy(hidx.at[1, pl.ds(base, J)], eiv_, sem_)

        def wait_stage(b, niv_, eiv_, sem_):
            base = w * R + b * J
            pltpu.make_async_copy(hidx.at[0, pl.ds(base, J)], niv_,
                                  sem_).wait()
            pltpu.make_async_copy(hidx.at[1, pl.ds(base, J)], eiv_,
                                  sem_).wait()

        def adds(niv_, eiv_):
            def fire(j, c2):
                pltpu.async_copy(ones_v, dv_sp.at[niv_.at[j]], sem_a, add=True)
                pltpu.async_copy(ones_v, de_sp.at[eiv_.at[j]], sem_b, add=True)
                return c2

            lax.fori_loop(0, J, fire, 0)

            def drain(j, c2):
                pltpu.make_async_copy(ones_v, dv_sp.at[niv_.at[j]],
                                      sem_a).wait()
                pltpu.make_async_copy(ones_v, de_sp.at[eiv_.at[j]],
                                      sem_b).wait()
                return c2

            lax.fori_loop(0, J, drain, 0)

        if nfull:
            stage(0, niv, eiv, sem_i)

        def outer2(h, c):
            b0 = 2 * h

            @pl.when(b0 + 1 < nfull)
            def _pre_b():
                stage(b0 + 1, niv2, eiv2, sem_i2)

            wait_stage(b0, niv, eiv, sem_i)
            adds(niv, eiv)

            @pl.when(b0 + 1 < nfull)
            def _do_b():
                @pl.when(b0 + 2 < nfull)
                def _pre_a():
                    stage(b0 + 2, niv, eiv, sem_i)

                wait_stage(b0 + 1, niv2, eiv2, sem_i2)
                adds(niv2, eiv2)

            return c

        lax.fori_loop(0, (nfull + 1) // 2, outer2, 0)

        if tail:
            def tail_body(t, c):
                row = w * R + nfull * J + t
                pltpu.sync_copy(hidx.at[0, pl.ds(row, 1)], niv.at[pl.ds(0, 1)])
                pltpu.sync_copy(hidx.at[1, pl.ds(row, 1)], eiv.at[pl.ds(0, 1)])
                pltpu.sync_copy(ones_v, dv_sp.at[niv.at[0]], add=True)
                pltpu.sync_copy(ones_v, de_sp.at[eiv.at[0]], add=True)
                return c

            lax.fori_loop(0, tail, tail_body, 0)

        if EXTRA:
            @pl.when(w < EXTRA)
            def _extra():
                row = NT * R + w
                pltpu.sync_copy(hidx.at[0, pl.ds(row, 1)], niv.at[pl.ds(0, 1)])
                pltpu.sync_copy(hidx.at[1, pl.ds(row, 1)], eiv.at[pl.ds(0, 1)])
                pltpu.sync_copy(ones_v, dv_sp.at[niv.at[0]], add=True)
                pltpu.sync_copy(ones_v, de_sp.at[eiv.at[0]], add=True)

        plsc.subcore_barrier()
        pltpu.sync_copy(dv_sp.at[pl.ds(sid * sl, sl)], zbuf)
        pltpu.sync_copy(zbuf, dv_out.at[pl.ds(cid * NP + sid * sl, sl)])
        pltpu.sync_copy(de_sp.at[pl.ds(sid * sl, sl)], zbuf)
        pltpu.sync_copy(zbuf, de_out.at[pl.ds(cid * NP + sid * sl, sl)])

    return pl.kernel(
        body,
        out_type=[
            jax.ShapeDtypeStruct((NC * NP,), jnp.float32),
            jax.ShapeDtypeStruct((NC * NP,), jnp.float32),
        ],
        mesh=mesh,
        scratch_types=[
            pltpu.VMEM_SHARED((NP,), jnp.float32),
            pltpu.VMEM_SHARED((NP,), jnp.float32),
            pltpu.VMEM((IDXW,), jnp.float32),
            pltpu.VMEM((sl,), jnp.float32),
            pltpu.VMEM((J, IDXW), jnp.int32),
            pltpu.VMEM((J, IDXW), jnp.int32),
            pltpu.VMEM((J, IDXW), jnp.int32),
            pltpu.VMEM((J, IDXW), jnp.int32),
            pltpu.SemaphoreType.DMA,
            pltpu.SemaphoreType.DMA,
            pltpu.SemaphoreType.DMA,
            pltpu.SemaphoreType.DMA,
        ],
        compiler_params=pltpu.CompilerParams(
            use_tc_tiling_on_sc=False, needs_layout_passes=False),
    )


def _rs_call(NPR):
    def body(dva, dea, rs, dvc, deinv):
        dv = jnp.clip(dva[0:NPR, :] + dva[NPR:2 * NPR, :], 1e-6, None)
        dvc[...] = dv
        rs[...] = lax.rsqrt(dv)
        deinv[...] = 1.0 / jnp.clip(
            dea[0:NPR, :] + dea[NPR:2 * NPR, :], 1e-6, None)

    return pl.pallas_call(
        body,
        out_shape=[
            jax.ShapeDtypeStruct((NPR, 128), jnp.float32),
            jax.ShapeDtypeStruct((NPR, 128), jnp.float32),
            jax.ShapeDtypeStruct((NPR, 128), jnp.float32),
        ],
    )


def _scatter_call(N, NP, K, ROWS, R, EXTRA, J):
    mesh = plsc.VectorSubcoreMesh(core_axis_name="c", subcore_axis_name="s")
    sl = NP // NS
    ZR = sl // 16
    nfull = R // J
    tail = R % J
    pairs = nfull // 2
    odd = nfull % 2
    CHT = ((N + NT - 1) // NT + LANES - 1) // LANES * LANES

    def body(rs_hbm, z_hbm, hidx, out_hbm, zn_hbm, acc_sp, zrow, nia, eia,
             nib, eib, nic, eic, nid, eid, buf_a, buf_b, rsb,
             sga, sgb, ssa, ssb, sgi, sgi2):
        cid = lax.axis_index("c")
        sid = lax.axis_index("s")
        w = cid * NS + sid

        def fill_zero(i, c):
            zrow[i] = jnp.zeros((LANES,), jnp.float32)
            return c

        lax.fori_loop(0, ZR, fill_zero, 0)
        for r in range(16):
            pltpu.sync_copy(zrow, acc_sp.at[pl.ds(sid * sl + r * ZR, ZR)])

        for h in range(2):
            start = pl.multiple_of(
                jnp.minimum(sid * 2 * CHT + h * CHT, N - CHT), 8)
            pltpu.sync_copy(z_hbm.at[pl.ds(start, CHT)],
                            buf_a.at[pl.ds(0, CHT)])
            pltpu.sync_copy(rs_hbm.at[pl.ds(start, CHT)], rsb)

            def ngrp(g, c):
                for r in range(LANES):
                    row = g * LANES + r
                    buf_a[row] = buf_a[row] * _splat(rsb, row)
                return c

            lax.fori_loop(0, CHT // LANES, ngrp, 0)
            pltpu.sync_copy(buf_a.at[pl.ds(0, CHT)],
                            zn_hbm.at[cid, pl.ds(start, CHT)])
        plsc.subcore_barrier()

        zn_c = zn_hbm.at[cid]

        def stage_idx(base, niv, eiv, sem_):
            pltpu.async_copy(hidx.at[0, pl.ds(base, J)], niv, sem_)
            pltpu.async_copy(hidx.at[1, pl.ds(base, J)], eiv, sem_)

        def wait_idx(base, niv, eiv, sem_):
            pltpu.make_async_copy(hidx.at[0, pl.ds(base, J)], niv, sem_).wait()
            pltpu.make_async_copy(hidx.at[1, pl.ds(base, J)], eiv, sem_).wait()

        def do_block(base, niv, eiv, buf, sg):
            def fire_g(j, c):
                pltpu.async_copy(zn_c.at[niv.at[j]],
                                 buf.at[pl.ds(j * IDXW, IDXW)], sg)
                return c

            lax.fori_loop(0, J, fire_g, 0)

        def drain_g_fire_s(niv, eiv, buf, sg, ss):
            def step(j, c):
                pltpu.make_async_copy(zn_c.at[niv.at[j]],
                                      buf.at[pl.ds(j * IDXW, IDXW)], sg).wait()
                pltpu.async_copy(buf.at[pl.ds(j * IDXW, IDXW)],
                                 acc_sp.at[eiv.at[j]], ss, add=True)
                return c

            lax.fori_loop(0, J, step, 0)

        def drain_s(eiv, buf, ss):
            def d(j, c):
                pltpu.make_async_copy(buf.at[pl.ds(j * IDXW, IDXW)],
                                      acc_sp.at[eiv.at[j]], ss).wait()
                return c

            lax.fori_loop(0, J, d, 0)

        def stage_pair(p, n0, e0, n1, e1, sem_):
            base0 = w * R + (2 * p) * J
            stage_idx(base0, n0, e0, sem_)
            stage_idx(base0 + J, n1, e1, sem_)

        def pair_proc(p, n0, e0, n1, e1, sem_):
            base0 = w * R + (2 * p) * J
            wait_idx(base0, n0, e0, sem_)
            do_block(base0, n0, e0, buf_a, sga)
            wait_idx(base0 + J, n1, e1, sem_)
            do_block(base0 + J, n1, e1, buf_b, sgb)
            drain_g_fire_s(n0, e0, buf_a, sga, ssa)
            drain_g_fire_s(n1, e1, buf_b, sgb, ssb)
            drain_s(e0, buf_a, ssa)
            drain_s(e1, buf_b, ssb)

        if pairs:
            stage_pair(0, nia, eia, nib, eib, sgi)

        def outer2(i, c):
            p0 = 2 * i

            @pl.when(p0 + 1 < pairs)
            def _pre_cd():
                stage_pair(p0 + 1, nic, eic, nid, eid, sgi2)

            pair_proc(p0, nia, eia, nib, eib, sgi)

            @pl.when(p0 + 1 < pairs)
            def _do_cd():
                @pl.when(p0 + 2 < pairs)
                def _pre_ab():
                    stage_pair(p0 + 2, nia, eia, nib, eib, sgi)

                pair_proc(p0 + 1, nic, eic, nid, eid, sgi2)

            return c

        lax.fori_loop(0, (pairs + 1) // 2, outer2, 0)

        if odd:
            base0 = w * R + (2 * pairs) * J
            stage_idx(base0, nia, eia, sgi)
            wait_idx(base0, nia, eia, sgi)
            do_block(base0, nia, eia, buf_a, sga)
            drain_g_fire_s(nia, eia, buf_a, sga, ssa)
            drain_s(eia, buf_a, ssa)

        def one_row(row):
            pltpu.sync_copy(hidx.at[0, pl.ds(row, 1)], nia.at[pl.ds(0, 1)])
            pltpu.sync_copy(hidx.at[1, pl.ds(row, 1)], eia.at[pl.ds(0, 1)])
            pltpu.async_copy(zn_c.at[nia.at[0]],
                             buf_a.at[pl.ds(0, IDXW)], sga).wait()
            pltpu.sync_copy(buf_a.at[pl.ds(0, IDXW)],
                            acc_sp.at[eia.at[0]], add=True)

        if tail:
            def tail_body(t, c):
                one_row(w * R + nfull * J + t)
                return c

            lax.fori_loop(0, tail, tail_body, 0)

        if EXTRA:
            @pl.when(w < EXTRA)
            def _extra():
                one_row(NT * R + w)

        plsc.subcore_barrier()
        for r in range(16):
            pltpu.sync_copy(acc_sp.at[pl.ds(sid * sl + r * ZR, ZR)], zrow)
            pltpu.sync_copy(zrow, out_hbm.at[cid, pl.ds(sid * sl + r * ZR, ZR)])

    return pl.kernel(
        body,
        out_type=[
            jax.ShapeDtypeStruct((NC, NP, K), jnp.float32),
            jax.ShapeDtypeStruct((NC, NP, K), jnp.float32),
        ],
        mesh=mesh,
        scratch_types=[
            pltpu.VMEM_SHARED((NP, K), jnp.float32),
            pltpu.VMEM((ZR, K), jnp.float32),
            pltpu.VMEM((J, IDXW), jnp.int32),
            pltpu.VMEM((J, IDXW), jnp.int32),
            pltpu.VMEM((J, IDXW), jnp.int32),
            pltpu.VMEM((J, IDXW), jnp.int32),
            pltpu.VMEM((J, IDXW), jnp.int32),
            pltpu.VMEM((J, IDXW), jnp.int32),
            pltpu.VMEM((J, IDXW), jnp.int32),
            pltpu.VMEM((J, IDXW), jnp.int32),
            pltpu.VMEM((J * IDXW, K), jnp.float32),
            pltpu.VMEM((J * IDXW, K), jnp.float32),
            pltpu.VMEM((CHT,), jnp.float32),
            pltpu.SemaphoreType.DMA,
            pltpu.SemaphoreType.DMA,
            pltpu.SemaphoreType.DMA,
            pltpu.SemaphoreType.DMA,
            pltpu.SemaphoreType.DMA,
            pltpu.SemaphoreType.DMA,
        ],
        compiler_params=pltpu.CompilerParams(
            use_tc_tiling_on_sc=False, needs_layout_passes=False),
    )


def _reduce_call(N, NP, K):
    mesh = plsc.VectorSubcoreMesh(core_axis_name="c", subcore_axis_name="s")
    CHT = ((N + NT - 1) // NT + LANES - 1) // LANES * LANES

    def body(wse, deinv_h, z_hbm, dvc_h, out_h, w0b, w1b, zb, dib, dcb,
             accv, idx32, part_sp, sem_s):
        cid = lax.axis_index("c")
        sid = lax.axis_index("s")
        w = cid * NS + sid

        for i in range(2):
            accv[pl.ds(i * LANES, LANES)] = jnp.zeros((LANES,), jnp.float32)
            idx32[pl.ds(i * LANES, LANES)] = (
                lax.iota(jnp.int32, LANES) + i * LANES)

        @pl.when(sid == 0)
        def _zero_part():
            pltpu.sync_copy(accv, part_sp)
        plsc.subcore_barrier()

        start = pl.multiple_of(
            jnp.maximum(jnp.minimum(w * CHT, N - CHT), 0), 8)
        off = w * CHT - start
        count = jnp.clip(N - w * CHT, 0, CHT)

        pltpu.async_copy(wse.at[0, pl.ds(start, CHT)], w0b, sem_s)
        pltpu.async_copy(wse.at[1, pl.ds(start, CHT)], w1b, sem_s)
        pltpu.async_copy(z_hbm.at[pl.ds(start, CHT)], zb, sem_s)
        pltpu.async_copy(deinv_h.at[pl.ds(start, CHT)], dib, sem_s)
        pltpu.async_copy(dvc_h.at[pl.ds(start, CHT)], dcb, sem_s)
        pltpu.make_async_copy(wse.at[0, pl.ds(start, CHT)], w0b, sem_s).wait()
        pltpu.make_async_copy(wse.at[1, pl.ds(start, CHT)], w1b, sem_s).wait()
        pltpu.make_async_copy(z_hbm.at[pl.ds(start, CHT)], zb, sem_s).wait()
        pltpu.make_async_copy(deinv_h.at[pl.ds(start, CHT)], dib, sem_s).wait()
        pltpu.make_async_copy(dvc_h.at[pl.ds(start, CHT)], dcb, sem_s).wait()

        def grp(g, carry):
            th, fd = carry
            base_row = off + g * LANES
            for r in range(LANES):
                row = base_row + r
                wr = w0b[row] + w1b[row]
                th = th + wr * wr * _splat(dib, row)
                zr = zb[row]
                fd = fd + zr * zr * _splat(dcb, row)
            return th, fd

        th, fd = lax.fori_loop(
            0, count // LANES, grp,
            (jnp.zeros((LANES,), jnp.float32), jnp.zeros((LANES,), jnp.float32)))
        accv[pl.ds(0, LANES)] = th
        accv[pl.ds(LANES, LANES)] = fd
        pltpu.sync_copy(accv, part_sp.at[idx32], add=True)
        plsc.subcore_barrier()

        @pl.when(sid == 0)
        def _out():
            pltpu.sync_copy(part_sp, accv)
            pltpu.sync_copy(accv, out_h.at[pl.ds(cid * 2 * K, 2 * K)])

    return pl.kernel(
        body,
        out_type=jax.ShapeDtypeStruct((NC * 2 * K,), jnp.float32),
        mesh=mesh,
        scratch_types=[
            pltpu.VMEM((CHT, K), jnp.float32),
            pltpu.VMEM((CHT, K), jnp.float32),
            pltpu.VMEM((CHT, K), jnp.float32),
            pltpu.VMEM((CHT,), jnp.float32),
            pltpu.VMEM((CHT,), jnp.float32),
            pltpu.VMEM((2 * K,), jnp.float32),
            pltpu.VMEM((2 * K,), jnp.int32),
            pltpu.VMEM_SHARED((2 * K,), jnp.float32),
            pltpu.SemaphoreType.DMA,
        ],
        compiler_params=pltpu.CompilerParams(
            use_tc_tiling_on_sc=False, needs_layout_passes=False),
    )


def _final_call(K):
    def body(p, out):
        v = p[...]
        theta = v[0:1, :] + v[2:3, :]
        fdvf = jnp.clip(v[1:2, :] + v[3:4, :], 1e-6, None)
        rq = 1.0 - theta / fdvf
        rq = jnp.where(jnp.isnan(rq) | jnp.isinf(rq), 0.0, rq)
        out[...] = jnp.mean(rq)[None, None]

    return pl.pallas_call(
        body,
        out_shape=jax.ShapeDtypeStruct((1, 1), jnp.float32),
    )


def kernel(Z, hyperedge_index, num_nodes):
    N, K = Z.shape
    E = hyperedge_index.shape[1]

    NP = ((N + 1 + 2047) // 2048) * 2048
    NPR = NP // 128

    if E % IDXW:
        epad = IDXW - E % IDXW
        hidx = jnp.concatenate(
            [hyperedge_index, jnp.full((2, epad), N, jnp.int32)], axis=1)
    else:
        hidx = hyperedge_index
    ROWS = hidx.shape[1] // IDXW
    hidx = hidx.reshape(2, ROWS, IDXW)
    R = ROWS // NT
    EXTRA = ROWS % NT
    JH = _pick_j(R, 40)
    JS = _pick_j(R, 13)

    dv_all, de_all = _hist_call(NP, ROWS, R, EXTRA, JH)(hidx)
    rs, dvc, deinv = _rs_call(NPR)(
        dv_all.reshape(2 * NPR, 128), de_all.reshape(2 * NPR, 128))
    rs = rs.reshape(NP)
    dvc = dvc.reshape(NP)
    deinv = deinv.reshape(NP)
    wse, _zn = _scatter_call(N, NP, K, ROWS, R, EXTRA, JS)(rs, Z, hidx)
    parts = _reduce_call(N, NP, K)(wse, deinv, Z, dvc)
    loss = _final_call(K)(parts.reshape(NC * 2, K))
    return loss[0, 0]
